# baseline (speedup 1.0000x reference)
import functools

import jax
import jax.numpy as jnp
from jax import lax
from jax.experimental import pallas as pl
from jax.experimental.pallas import tpu as pltpu
from jax.experimental.pallas import tpu_sc as plsc

E = 8
D_MODEL = 1024
D_HIDDEN = 512
N = 4096

TM = 256
NT = N // TM
T_VISITS = NT + E - 1
TM2 = 512

SC_NC = 2
SC_NS = 16
NW = SC_NC * SC_NS
ROWS_PER_W = N // NW
CH = 32
NCH = ROWS_PER_W // CH


def _sc_gather(table, idx3d, d, dtype):
  mesh = plsc.VectorSubcoreMesh(core_axis_name="c", subcore_axis_name="s")

  @functools.partial(
      pl.kernel,
      mesh=mesh,
      out_type=jax.ShapeDtypeStruct((N, d), dtype),
      scratch_types=[
          pltpu.VMEM((NCH, CH), jnp.int32),
          pltpu.VMEM((CH, d), dtype),
          pltpu.VMEM((CH, d), dtype),
          pltpu.SemaphoreType.DMA,
          pltpu.SemaphoreType.DMA,
      ],
  )
  def gather_kernel(table_hbm, idx_hbm, out_hbm, idx_v, buf0, buf1, sem0, sem1):
    wid = lax.axis_index("s") * SC_NC + lax.axis_index("c")
    base = wid * ROWS_PER_W
    pltpu.sync_copy(idx_hbm.at[wid], idx_v)
    bufs = (buf0, buf1)
    sems = (sem0, sem1)
    handles = [None, None]
    handles[0] = pltpu.async_copy(table_hbm.at[idx_v.at[0]], buf0, sem0)
    for c in range(NCH):
      if c + 1 < NCH:
        handles[(c + 1) % 2] = pltpu.async_copy(
            table_hbm.at[idx_v.at[c + 1]], bufs[(c + 1) % 2], sems[(c + 1) % 2])
      handles[c % 2].wait()
      pltpu.sync_copy(bufs[c % 2], out_hbm.at[pl.ds(base + c * CH, CH)])

  return gather_kernel(table, idx3d)


def _sc_scatter(table, idx3d, d, dtype):
  mesh = plsc.VectorSubcoreMesh(core_axis_name="c", subcore_axis_name="s")

  @functools.partial(
      pl.kernel,
      mesh=mesh,
      out_type=jax.ShapeDtypeStruct((N, d), dtype),
      scratch_types=[
          pltpu.VMEM((NCH, CH), jnp.int32),
          pltpu.VMEM((CH, d), dtype),
          pltpu.VMEM((CH, d), dtype),
          pltpu.SemaphoreType.DMA,
          pltpu.SemaphoreType.DMA,
          pltpu.SemaphoreType.DMA,
      ],
  )
  def scatter_kernel(table_hbm, idx_hbm, out_hbm, idx_v, buf0, buf1,
                     sem0, sem1, wsem):
    wid = lax.axis_index("s") * SC_NC + lax.axis_index("c")
    base = wid * ROWS_PER_W
    pltpu.sync_copy(idx_hbm.at[wid], idx_v)
    bufs = (buf0, buf1)
    sems = (sem0, sem1)
    handles = [None, None]
    handles[0] = pltpu.async_copy(table_hbm.at[pl.ds(base, CH)], buf0, sem0)
    wh = [None, None]
    for c in range(NCH):
      if c + 1 < NCH:
        if wh[(c + 1) % 2] is not None:
          wh[(c + 1) % 2].wait()
        handles[(c + 1) % 2] = pltpu.async_copy(
            table_hbm.at[pl.ds(base + (c + 1) * CH, CH)],
            bufs[(c + 1) % 2], sems[(c + 1) % 2])
      handles[c % 2].wait()
      wh[c % 2] = pltpu.async_copy(bufs[c % 2], out_hbm.at[idx_v.at[c]], wsem)
    for h in wh:
      if h is not None:
        h.wait()

  return scatter_kernel(table, idx3d)


def _grouped_encode(tile_ids, group_ids, seg_starts, seg_ends,
                    x_sorted, W_enc, b_enc, W_dec, b_dec_r):
  def body(tids, gids, st, en, x_ref, we_ref, be_ref, wd_ref, bd_ref,
           out_ref, acc_ref, web_ref, wdb_ref, lacc_ref):
    t = pl.program_id(0)

    first_g = (t == 0) | (gids[t] != gids[jnp.maximum(t - 1, 0)])

    @pl.when(first_g)
    def _():
      web_ref[...] = we_ref[0].astype(jnp.bfloat16)

    @pl.when(t == 0)
    def _():
      wdb_ref[...] = wd_ref[...].astype(jnp.bfloat16)
      lacc_ref[...] = jnp.zeros((1, D_MODEL), jnp.float32)

    x = x_ref[...]
    enc = jnp.dot(x.astype(jnp.bfloat16), web_ref[...],
                  preferred_element_type=jnp.float32)
    enc = jnp.maximum(enc + be_ref[0, 0], 0.0).astype(jnp.bfloat16)
    h = D_HIDDEN // 2
    e0 = pltpu.bitcast(enc[:, :h], jnp.uint16).astype(jnp.uint32)
    e1 = pltpu.bitcast(enc[:, h:], jnp.uint16).astype(jnp.uint32)
    packed = pltpu.bitcast(e0 | (e1 << 16), jnp.int32)

    base = tids[t] * TM
    ri = base + lax.broadcasted_iota(jnp.int32, (TM, 1), 0)
    mask = (ri >= st[t]) & (ri < en[t])
    out_ref[...] = jnp.where(mask, packed, out_ref[...])

    dec = jnp.dot(enc, wdb_ref[...],
                  preferred_element_type=jnp.float32) + bd_ref[0]
    diff = dec - x
    diff2 = jnp.where(mask, diff * diff, 0.0)
    lacc_ref[...] += jnp.sum(diff2, axis=0, keepdims=True)

    @pl.when(t == T_VISITS - 1)
    def _():
      acc_ref[0, 0] = jnp.sum(lacc_ref[...]) * (1.0 / (N * D_MODEL))

  grid_spec = pltpu.PrefetchScalarGridSpec(
      num_scalar_prefetch=4,
      grid=(T_VISITS,),
      in_specs=[
          pl.BlockSpec((TM, D_MODEL), lambda t, tids, gids, st, en: (tids[t], 0)),
          pl.BlockSpec((1, D_MODEL, D_HIDDEN),
                       lambda t, tids, gids, st, en: (gids[t], 0, 0)),
          pl.BlockSpec((1, 1, D_HIDDEN),
                       lambda t, tids, gids, st, en: (gids[t], 0, 0)),
          pl.BlockSpec((D_HIDDEN, D_MODEL),
                       lambda t, tids, gids, st, en: (0, 0)),
          pl.BlockSpec((1, D_MODEL), lambda t, tids, gids, st, en: (0, 0)),
      ],
      out_specs=[
          pl.BlockSpec((TM, D_HIDDEN // 2),
                       lambda t, tids, gids, st, en: (tids[t], 0)),
          pl.BlockSpec(memory_space=pltpu.SMEM),
      ],
      scratch_shapes=[
          pltpu.VMEM((D_MODEL, D_HIDDEN), jnp.bfloat16),
          pltpu.VMEM((D_HIDDEN, D_MODEL), jnp.bfloat16),
          pltpu.VMEM((1, D_MODEL), jnp.float32),
      ],
  )

  return pl.pallas_call(
      body,
      grid_spec=grid_spec,
      out_shape=[
          jax.ShapeDtypeStruct((N, D_HIDDEN // 2), jnp.int32),
          jax.ShapeDtypeStruct((1, 1), jnp.float32),
      ],
      compiler_params=pltpu.CompilerParams(
          dimension_semantics=("arbitrary",)),
  )(tile_ids, group_ids, seg_starts, seg_ends,
    x_sorted, W_enc, b_enc.reshape(E, 1, D_HIDDEN), W_dec, b_dec_r)


def _decode(enc, W_dec, b_dec_r):
  nt2 = N // TM2

  def body(enc_ref, wd_ref, bd_ref, out_ref, wdb_ref):
    t = pl.program_id(0)

    @pl.when(t == 0)
    def _():
      wdb_ref[...] = wd_ref[...].astype(jnp.bfloat16)

    h = D_HIDDEN // 2
    w = pltpu.bitcast(enc_ref[...], jnp.uint32)
    e0 = pltpu.bitcast((w & 0xFFFF).astype(jnp.uint16), jnp.bfloat16)
    e1 = pltpu.bitcast((w >> 16).astype(jnp.uint16), jnp.bfloat16)
    out_ref[...] = (
        jnp.dot(e0, wdb_ref[:h], preferred_element_type=jnp.float32)
        + jnp.dot(e1, wdb_ref[h:], preferred_element_type=jnp.float32)
        + bd_ref[0])

  return pl.pallas_call(
      body,
      grid=(nt2,),
      in_specs=[
          pl.BlockSpec((TM2, D_HIDDEN // 2), lambda t: (t, 0)),
          pl.BlockSpec((D_HIDDEN, D_MODEL), lambda t: (0, 0)),
          pl.BlockSpec((1, D_MODEL), lambda t: (0, 0)),
      ],
      out_specs=pl.BlockSpec((TM2, D_MODEL), lambda t: (t, 0)),
      out_shape=jax.ShapeDtypeStruct((N, D_MODEL), jnp.float32),
      scratch_shapes=[
          pltpu.VMEM((D_HIDDEN, D_MODEL), jnp.bfloat16),
      ],
      compiler_params=pltpu.CompilerParams(
          dimension_semantics=("arbitrary",)),
  )(enc, W_dec, b_dec_r)


def kernel(img, label, W_enc, b_enc, W_dec, b_dec):
  label = label.astype(jnp.int32)

  oh = (label[:, None] == jnp.arange(E, dtype=jnp.int32)[None, :]).astype(
      jnp.int32)
  csum = jnp.cumsum(oh, axis=0)
  sizes = csum[-1]
  ends = jnp.cumsum(sizes)
  starts = ends - sizes
  within = jnp.sum(oh * csum, axis=1) - 1
  rank = jnp.sum(oh * starts[None, :], axis=1) + within
  rank3d = rank.reshape(NW, NCH, CH)
  nonzero = sizes > 0
  first_tile = starts // TM
  last_tile = jnp.where(nonzero, (ends - 1) // TM, first_tile)
  ntiles = jnp.where(nonzero, last_tile - first_tile + 1, 0)
  cum = jnp.cumsum(ntiles)
  cum_ex = cum - ntiles
  n_visits = cum[E - 1]

  t_idx = jnp.arange(T_VISITS, dtype=jnp.int32)
  e_of_t = jnp.minimum(
      jnp.searchsorted(cum, t_idx, side="right").astype(jnp.int32), E - 1)
  valid = t_idx < n_visits
  tile_ids = jnp.where(valid, first_tile[e_of_t] + t_idx - cum_ex[e_of_t],
                       NT - 1).astype(jnp.int32)
  group_ids = jnp.where(valid, e_of_t, 0).astype(jnp.int32)
  seg_starts = jnp.where(valid, starts[e_of_t], 0).astype(jnp.int32)
  seg_ends = jnp.where(valid, ends[e_of_t], 0).astype(jnp.int32)

  x_sorted = _sc_scatter(img, rank3d, D_MODEL, jnp.float32)

  enc_sorted, loss_sum = _grouped_encode(
      tile_ids, group_ids, seg_starts, seg_ends,
      x_sorted, W_enc, b_enc, W_dec, b_dec.reshape(1, D_MODEL))

  enc = _sc_gather(enc_sorted, rank3d, D_HIDDEN // 2, jnp.int32)

  decoded = _decode(enc, W_dec, b_dec.reshape(1, D_MODEL))

  return (loss_sum[0, 0], decoded)


# speedup vs baseline: 1.0273x; 1.0273x over previous
import functools

import jax
import jax.numpy as jnp
from jax import lax
from jax.experimental import pallas as pl
from jax.experimental.pallas import tpu as pltpu
from jax.experimental.pallas import tpu_sc as plsc

E = 8
D_MODEL = 1024
D_HIDDEN = 512
N = 4096

TM = 256
NT = N // TM
T_VISITS = NT + E - 1
TM2 = 512

SC_NC = 2
SC_NS = 16
NW = SC_NC * SC_NS
ROWS_PER_W = N // NW
CH = 32
NCH = ROWS_PER_W // CH


def _sc_gather(table, idx3d, d, dtype):
  mesh = plsc.VectorSubcoreMesh(core_axis_name="c", subcore_axis_name="s")

  @functools.partial(
      pl.kernel,
      mesh=mesh,
      out_type=jax.ShapeDtypeStruct((N, d), dtype),
      scratch_types=[
          pltpu.VMEM((NCH, CH), jnp.int32),
          pltpu.VMEM((CH, d), dtype),
          pltpu.VMEM((CH, d), dtype),
          pltpu.VMEM((CH, d), dtype),
          pltpu.SemaphoreType.DMA,
          pltpu.SemaphoreType.DMA,
          pltpu.SemaphoreType.DMA,
          pltpu.SemaphoreType.DMA,
      ],
  )
  def gather_kernel(table_hbm, idx_hbm, out_hbm, idx_v, buf0, buf1, buf2,
                    sem0, sem1, sem2, wsem):
    wid = lax.axis_index("s") * SC_NC + lax.axis_index("c")
    base = wid * ROWS_PER_W
    pltpu.sync_copy(idx_hbm.at[wid], idx_v)
    bufs = (buf0, buf1, buf2)
    sems = (sem0, sem1, sem2)
    rh = [None, None, None]
    wh = [None, None, None]
    for c in range(min(3, NCH)):
      rh[c] = pltpu.async_copy(table_hbm.at[idx_v.at[c]], bufs[c], sems[c])
    for c in range(NCH):
      b = c % 3
      if c >= 3:
        wh[b].wait()
        rh[b] = pltpu.async_copy(table_hbm.at[idx_v.at[c]], bufs[b], sems[b])
      rh[b].wait()
      wh[b] = pltpu.async_copy(bufs[b], out_hbm.at[pl.ds(base + c * CH, CH)],
                               wsem)
    for c in range(max(0, NCH - 3), NCH):
      wh[c % 3].wait()

  return gather_kernel(table, idx3d)


def _sc_scatter(table, idx3d, d, dtype):
  mesh = plsc.VectorSubcoreMesh(core_axis_name="c", subcore_axis_name="s")

  @functools.partial(
      pl.kernel,
      mesh=mesh,
      out_type=jax.ShapeDtypeStruct((N, d), dtype),
      scratch_types=[
          pltpu.VMEM((NCH, CH), jnp.int32),
          pltpu.VMEM((CH, d), dtype),
          pltpu.VMEM((CH, d), dtype),
          pltpu.VMEM((CH, d), dtype),
          pltpu.SemaphoreType.DMA,
          pltpu.SemaphoreType.DMA,
          pltpu.SemaphoreType.DMA,
          pltpu.SemaphoreType.DMA,
      ],
  )
  def scatter_kernel(table_hbm, idx_hbm, out_hbm, idx_v, buf0, buf1, buf2,
                     sem0, sem1, sem2, wsem):
    wid = lax.axis_index("s") * SC_NC + lax.axis_index("c")
    base = wid * ROWS_PER_W
    pltpu.sync_copy(idx_hbm.at[wid], idx_v)
    bufs = (buf0, buf1, buf2)
    sems = (sem0, sem1, sem2)
    rh = [None, None, None]
    wh = [None, None, None]
    for c in range(min(3, NCH)):
      rh[c] = pltpu.async_copy(table_hbm.at[pl.ds(base + c * CH, CH)],
                               bufs[c], sems[c])
    for c in range(NCH):
      b = c % 3
      if c >= 3:
        wh[b].wait()
        rh[b] = pltpu.async_copy(table_hbm.at[pl.ds(base + c * CH, CH)],
                                 bufs[b], sems[b])
      rh[b].wait()
      wh[b] = pltpu.async_copy(bufs[b], out_hbm.at[idx_v.at[c]], wsem)
    for c in range(max(0, NCH - 3), NCH):
      wh[c % 3].wait()

  return scatter_kernel(table, idx3d)


def _grouped_encode(tile_ids, group_ids, seg_starts, seg_ends,
                    x_sorted, W_enc, b_enc):
  def body(tids, gids, st, en, x_ref, we_ref, be_ref,
           out_ref, web_ref):
    t = pl.program_id(0)

    first_g = (t == 0) | (gids[t] != gids[jnp.maximum(t - 1, 0)])

    @pl.when(first_g)
    def _():
      web_ref[...] = we_ref[0].astype(jnp.bfloat16)

    x = x_ref[...]
    enc = jnp.dot(x.astype(jnp.bfloat16), web_ref[...],
                  preferred_element_type=jnp.float32)
    enc = jnp.maximum(enc + be_ref[0, 0], 0.0).astype(jnp.bfloat16)
    h = D_HIDDEN // 2
    e0 = pltpu.bitcast(enc[:, :h], jnp.uint16).astype(jnp.uint32)
    e1 = pltpu.bitcast(enc[:, h:], jnp.uint16).astype(jnp.uint32)
    packed = pltpu.bitcast(e0 | (e1 << 16), jnp.int32)

    base = tids[t] * TM
    ri = base + lax.broadcasted_iota(jnp.int32, (TM, 1), 0)
    mask = (ri >= st[t]) & (ri < en[t])
    out_ref[...] = jnp.where(mask, packed, out_ref[...])

  grid_spec = pltpu.PrefetchScalarGridSpec(
      num_scalar_prefetch=4,
      grid=(T_VISITS,),
      in_specs=[
          pl.BlockSpec((TM, D_MODEL), lambda t, tids, gids, st, en: (tids[t], 0)),
          pl.BlockSpec((1, D_MODEL, D_HIDDEN),
                       lambda t, tids, gids, st, en: (gids[t], 0, 0)),
          pl.BlockSpec((1, 1, D_HIDDEN),
                       lambda t, tids, gids, st, en: (gids[t], 0, 0)),
      ],
      out_specs=pl.BlockSpec((TM, D_HIDDEN // 2),
                             lambda t, tids, gids, st, en: (tids[t], 0)),
      scratch_shapes=[
          pltpu.VMEM((D_MODEL, D_HIDDEN), jnp.bfloat16),
      ],
  )

  return pl.pallas_call(
      body,
      grid_spec=grid_spec,
      out_shape=jax.ShapeDtypeStruct((N, D_HIDDEN // 2), jnp.int32),
      compiler_params=pltpu.CompilerParams(
          dimension_semantics=("arbitrary",)),
  )(tile_ids, group_ids, seg_starts, seg_ends,
    x_sorted, W_enc, b_enc.reshape(E, 1, D_HIDDEN))


def _decode_loss(enc, img, W_dec, b_dec_r):
  nt2 = N // TM2

  def body(enc_ref, x_ref, wd_ref, bd_ref, out_ref, acc_ref, wdb_ref,
           lacc_ref):
    t = pl.program_id(0)

    @pl.when(t == 0)
    def _():
      wdb_ref[...] = wd_ref[...].astype(jnp.bfloat16)
      lacc_ref[...] = jnp.zeros((1, D_MODEL), jnp.float32)

    h = D_HIDDEN // 2
    w = pltpu.bitcast(enc_ref[...], jnp.uint32)
    e0 = pltpu.bitcast((w & 0xFFFF).astype(jnp.uint16), jnp.bfloat16)
    e1 = pltpu.bitcast((w >> 16).astype(jnp.uint16), jnp.bfloat16)
    dec = (jnp.dot(e0, wdb_ref[:h], preferred_element_type=jnp.float32)
           + jnp.dot(e1, wdb_ref[h:], preferred_element_type=jnp.float32)
           + bd_ref[0])
    out_ref[...] = dec
    diff = dec - x_ref[...]
    lacc_ref[...] += jnp.sum(diff * diff, axis=0, keepdims=True)

    @pl.when(t == nt2 - 1)
    def _():
      acc_ref[0, 0] = jnp.sum(lacc_ref[...]) * (1.0 / (N * D_MODEL))

  return pl.pallas_call(
      body,
      grid=(nt2,),
      in_specs=[
          pl.BlockSpec((TM2, D_HIDDEN // 2), lambda t: (t, 0)),
          pl.BlockSpec((TM2, D_MODEL), lambda t: (t, 0)),
          pl.BlockSpec((D_HIDDEN, D_MODEL), lambda t: (0, 0)),
          pl.BlockSpec((1, D_MODEL), lambda t: (0, 0)),
      ],
      out_specs=[
          pl.BlockSpec((TM2, D_MODEL), lambda t: (t, 0)),
          pl.BlockSpec(memory_space=pltpu.SMEM),
      ],
      out_shape=[
          jax.ShapeDtypeStruct((N, D_MODEL), jnp.float32),
          jax.ShapeDtypeStruct((1, 1), jnp.float32),
      ],
      scratch_shapes=[
          pltpu.VMEM((D_HIDDEN, D_MODEL), jnp.bfloat16),
          pltpu.VMEM((1, D_MODEL), jnp.float32),
      ],
      compiler_params=pltpu.CompilerParams(
          dimension_semantics=("arbitrary",)),
  )(enc, img, W_dec, b_dec_r)


def kernel(img, label, W_enc, b_enc, W_dec, b_dec):
  label = label.astype(jnp.int32)

  oh = (label[:, None] == jnp.arange(E, dtype=jnp.int32)[None, :]).astype(
      jnp.int32)
  csum = jnp.cumsum(oh, axis=0)
  sizes = csum[-1]
  ends = jnp.cumsum(sizes)
  starts = ends - sizes
  within = jnp.sum(oh * csum, axis=1) - 1
  rank = jnp.sum(oh * starts[None, :], axis=1) + within
  rank3d = rank.reshape(NW, NCH, CH)
  nonzero = sizes > 0
  first_tile = starts // TM
  last_tile = jnp.where(nonzero, (ends - 1) // TM, first_tile)
  ntiles = jnp.where(nonzero, last_tile - first_tile + 1, 0)
  cum = jnp.cumsum(ntiles)
  cum_ex = cum - ntiles
  n_visits = cum[E - 1]

  t_idx = jnp.arange(T_VISITS, dtype=jnp.int32)
  e_of_t = jnp.minimum(
      jnp.searchsorted(cum, t_idx, side="right").astype(jnp.int32), E - 1)
  valid = t_idx < n_visits
  tile_ids = jnp.where(valid, first_tile[e_of_t] + t_idx - cum_ex[e_of_t],
                       NT - 1).astype(jnp.int32)
  group_ids = jnp.where(valid, e_of_t, 0).astype(jnp.int32)
  seg_starts = jnp.where(valid, starts[e_of_t], 0).astype(jnp.int32)
  seg_ends = jnp.where(valid, ends[e_of_t], 0).astype(jnp.int32)

  x_sorted = _sc_scatter(img, rank3d, D_MODEL, jnp.float32)

  enc_sorted = _grouped_encode(tile_ids, group_ids, seg_starts, seg_ends,
                               x_sorted, W_enc, b_enc)

  enc = _sc_gather(enc_sorted, rank3d, D_HIDDEN // 2, jnp.int32)

  decoded, loss_sum = _decode_loss(enc, img, W_dec,
                                   b_dec.reshape(1, D_MODEL))

  return (loss_sum[0, 0], decoded)


# speedup vs baseline: 1.0899x; 1.0609x over previous
import functools

import jax
import jax.numpy as jnp
from jax import lax
from jax.experimental import pallas as pl
from jax.experimental.pallas import tpu as pltpu
from jax.experimental.pallas import tpu_sc as plsc

E = 8
D_MODEL = 1024
D_HIDDEN = 512
N = 4096

TM = 512
NT = N // TM
T_VISITS = NT + E - 1
TM2 = 1024

SC_NC = 2
SC_NS = 16
NW = SC_NC * SC_NS
ROWS_PER_W = N // NW
CH = 32
NCH = ROWS_PER_W // CH


def _sc_gather(table, idx3d, d, dtype):
  mesh = plsc.VectorSubcoreMesh(core_axis_name="c", subcore_axis_name="s")

  @functools.partial(
      pl.kernel,
      mesh=mesh,
      out_type=jax.ShapeDtypeStruct((N, d), dtype),
      scratch_types=[
          pltpu.VMEM((NCH, CH), jnp.int32),
          pltpu.VMEM((CH, d), dtype),
          pltpu.VMEM((CH, d), dtype),
          pltpu.VMEM((CH, d), dtype),
          pltpu.SemaphoreType.DMA,
          pltpu.SemaphoreType.DMA,
          pltpu.SemaphoreType.DMA,
          pltpu.SemaphoreType.DMA,
      ],
  )
  def gather_kernel(table_hbm, idx_hbm, out_hbm, idx_v, buf0, buf1, buf2,
                    sem0, sem1, sem2, wsem):
    wid = lax.axis_index("s") * SC_NC + lax.axis_index("c")
    base = wid * ROWS_PER_W
    pltpu.sync_copy(idx_hbm.at[wid], idx_v)
    bufs = (buf0, buf1, buf2)
    sems = (sem0, sem1, sem2)
    rh = [None, None, None]
    wh = [None, None, None]
    for c in range(min(3, NCH)):
      rh[c] = pltpu.async_copy(table_hbm.at[idx_v.at[c]], bufs[c], sems[c])
    for c in range(NCH):
      b = c % 3
      if c >= 3:
        wh[b].wait()
        rh[b] = pltpu.async_copy(table_hbm.at[idx_v.at[c]], bufs[b], sems[b])
      rh[b].wait()
      wh[b] = pltpu.async_copy(bufs[b], out_hbm.at[pl.ds(base + c * CH, CH)],
                               wsem)
    for c in range(max(0, NCH - 3), NCH):
      wh[c % 3].wait()

  return gather_kernel(table, idx3d)


def _sc_scatter(table, idx3d, d, dtype):
  mesh = plsc.VectorSubcoreMesh(core_axis_name="c", subcore_axis_name="s")

  @functools.partial(
      pl.kernel,
      mesh=mesh,
      out_type=jax.ShapeDtypeStruct((N, d), dtype),
      scratch_types=[
          pltpu.VMEM((NCH, CH), jnp.int32),
          pltpu.VMEM((CH, d), dtype),
          pltpu.VMEM((CH, d), dtype),
          pltpu.VMEM((CH, d), dtype),
          pltpu.SemaphoreType.DMA,
          pltpu.SemaphoreType.DMA,
          pltpu.SemaphoreType.DMA,
          pltpu.SemaphoreType.DMA,
      ],
  )
  def scatter_kernel(table_hbm, idx_hbm, out_hbm, idx_v, buf0, buf1, buf2,
                     sem0, sem1, sem2, wsem):
    wid = lax.axis_index("s") * SC_NC + lax.axis_index("c")
    base = wid * ROWS_PER_W
    pltpu.sync_copy(idx_hbm.at[wid], idx_v)
    bufs = (buf0, buf1, buf2)
    sems = (sem0, sem1, sem2)
    rh = [None, None, None]
    wh = [None, None, None]
    for c in range(min(3, NCH)):
      rh[c] = pltpu.async_copy(table_hbm.at[pl.ds(base + c * CH, CH)],
                               bufs[c], sems[c])
    for c in range(NCH):
      b = c % 3
      if c >= 3:
        wh[b].wait()
        rh[b] = pltpu.async_copy(table_hbm.at[pl.ds(base + c * CH, CH)],
                                 bufs[b], sems[b])
      rh[b].wait()
      wh[b] = pltpu.async_copy(bufs[b], out_hbm.at[idx_v.at[c]], wsem)
    for c in range(max(0, NCH - 3), NCH):
      wh[c % 3].wait()

  return scatter_kernel(table, idx3d)


def _grouped_encode(tile_ids, group_ids, seg_starts, seg_ends,
                    x_sorted, W_enc, b_enc):
  def body(tids, gids, st, en, x_ref, we_ref, be_ref,
           out_ref, web_ref):
    t = pl.program_id(0)

    first_g = (t == 0) | (gids[t] != gids[jnp.maximum(t - 1, 0)])

    @pl.when(first_g)
    def _():
      web_ref[...] = we_ref[0].astype(jnp.bfloat16)

    x = x_ref[...]
    enc = jnp.dot(x.astype(jnp.bfloat16), web_ref[...],
                  preferred_element_type=jnp.float32)
    enc = jnp.maximum(enc + be_ref[0, 0], 0.0).astype(jnp.bfloat16)
    h = D_HIDDEN // 2
    e0 = pltpu.bitcast(enc[:, :h], jnp.uint16).astype(jnp.uint32)
    e1 = pltpu.bitcast(enc[:, h:], jnp.uint16).astype(jnp.uint32)
    packed = pltpu.bitcast(e0 | (e1 << 16), jnp.int32)

    base = tids[t] * TM
    ri = base + lax.broadcasted_iota(jnp.int32, (TM, 1), 0)
    mask = (ri >= st[t]) & (ri < en[t])
    out_ref[...] = jnp.where(mask, packed, out_ref[...])

  grid_spec = pltpu.PrefetchScalarGridSpec(
      num_scalar_prefetch=4,
      grid=(T_VISITS,),
      in_specs=[
          pl.BlockSpec((TM, D_MODEL), lambda t, tids, gids, st, en: (tids[t], 0)),
          pl.BlockSpec((1, D_MODEL, D_HIDDEN),
                       lambda t, tids, gids, st, en: (gids[t], 0, 0)),
          pl.BlockSpec((1, 1, D_HIDDEN),
                       lambda t, tids, gids, st, en: (gids[t], 0, 0)),
      ],
      out_specs=pl.BlockSpec((TM, D_HIDDEN // 2),
                             lambda t, tids, gids, st, en: (tids[t], 0)),
      scratch_shapes=[
          pltpu.VMEM((D_MODEL, D_HIDDEN), jnp.bfloat16),
      ],
  )

  return pl.pallas_call(
      body,
      grid_spec=grid_spec,
      out_shape=jax.ShapeDtypeStruct((N, D_HIDDEN // 2), jnp.int32),
      compiler_params=pltpu.CompilerParams(
          dimension_semantics=("arbitrary",)),
  )(tile_ids, group_ids, seg_starts, seg_ends,
    x_sorted, W_enc, b_enc.reshape(E, 1, D_HIDDEN))


def _decode_loss(enc, img, W_dec, b_dec_r):
  nt2 = N // TM2

  def body(enc_ref, x_ref, wd_ref, bd_ref, out_ref, acc_ref, wdb_ref,
           lacc_ref):
    t = pl.program_id(0)

    @pl.when(t == 0)
    def _():
      wdb_ref[...] = wd_ref[...].astype(jnp.bfloat16)
      lacc_ref[...] = jnp.zeros((1, D_MODEL), jnp.float32)

    h = D_HIDDEN // 2
    w = pltpu.bitcast(enc_ref[...], jnp.uint32)
    e0 = pltpu.bitcast((w & 0xFFFF).astype(jnp.uint16), jnp.bfloat16)
    e1 = pltpu.bitcast((w >> 16).astype(jnp.uint16), jnp.bfloat16)
    dec = (jnp.dot(e0, wdb_ref[:h], preferred_element_type=jnp.float32)
           + jnp.dot(e1, wdb_ref[h:], preferred_element_type=jnp.float32)
           + bd_ref[0])
    out_ref[...] = dec
    diff = dec - x_ref[...]
    lacc_ref[...] += jnp.sum(diff * diff, axis=0, keepdims=True)

    @pl.when(t == nt2 - 1)
    def _():
      acc_ref[0, 0] = jnp.sum(lacc_ref[...]) * (1.0 / (N * D_MODEL))

  return pl.pallas_call(
      body,
      grid=(nt2,),
      in_specs=[
          pl.BlockSpec((TM2, D_HIDDEN // 2), lambda t: (t, 0)),
          pl.BlockSpec((TM2, D_MODEL), lambda t: (t, 0)),
          pl.BlockSpec((D_HIDDEN, D_MODEL), lambda t: (0, 0)),
          pl.BlockSpec((1, D_MODEL), lambda t: (0, 0)),
      ],
      out_specs=[
          pl.BlockSpec((TM2, D_MODEL), lambda t: (t, 0)),
          pl.BlockSpec(memory_space=pltpu.SMEM),
      ],
      out_shape=[
          jax.ShapeDtypeStruct((N, D_MODEL), jnp.float32),
          jax.ShapeDtypeStruct((1, 1), jnp.float32),
      ],
      scratch_shapes=[
          pltpu.VMEM((D_HIDDEN, D_MODEL), jnp.bfloat16),
          pltpu.VMEM((1, D_MODEL), jnp.float32),
      ],
      compiler_params=pltpu.CompilerParams(
          dimension_semantics=("arbitrary",)),
  )(enc, img, W_dec, b_dec_r)


def kernel(img, label, W_enc, b_enc, W_dec, b_dec):
  label = label.astype(jnp.int32)

  oh = (label[:, None] == jnp.arange(E, dtype=jnp.int32)[None, :]).astype(
      jnp.int32)
  csum = jnp.cumsum(oh, axis=0)
  sizes = csum[-1]
  ends = jnp.cumsum(sizes)
  starts = ends - sizes
  within = jnp.sum(oh * csum, axis=1) - 1
  rank = jnp.sum(oh * starts[None, :], axis=1) + within
  rank3d = rank.reshape(NW, NCH, CH)
  nonzero = sizes > 0
  first_tile = starts // TM
  last_tile = jnp.where(nonzero, (ends - 1) // TM, first_tile)
  ntiles = jnp.where(nonzero, last_tile - first_tile + 1, 0)
  cum = jnp.cumsum(ntiles)
  cum_ex = cum - ntiles
  n_visits = cum[E - 1]

  t_idx = jnp.arange(T_VISITS, dtype=jnp.int32)
  e_of_t = jnp.minimum(
      jnp.searchsorted(cum, t_idx, side="right").astype(jnp.int32), E - 1)
  valid = t_idx < n_visits
  tile_ids = jnp.where(valid, first_tile[e_of_t] + t_idx - cum_ex[e_of_t],
                       NT - 1).astype(jnp.int32)
  group_ids = jnp.where(valid, e_of_t, 0).astype(jnp.int32)
  seg_starts = jnp.where(valid, starts[e_of_t], 0).astype(jnp.int32)
  seg_ends = jnp.where(valid, ends[e_of_t], 0).astype(jnp.int32)

  x_sorted = _sc_scatter(img, rank3d, D_MODEL, jnp.float32)

  enc_sorted = _grouped_encode(tile_ids, group_ids, seg_starts, seg_ends,
                               x_sorted, W_enc, b_enc)

  enc = _sc_gather(enc_sorted, rank3d, D_HIDDEN // 2, jnp.int32)

  decoded, loss_sum = _decode_loss(enc, img, W_dec,
                                   b_dec.reshape(1, D_MODEL))

  return (loss_sum[0, 0], decoded)
